# baseline (speedup 1.0000x reference)
import functools
import math

import jax
import jax.numpy as jnp
from jax.experimental import pallas as pl
from jax.experimental.pallas import tpu as pltpu

Q = 1024
D_IN = 512
D_EMB = 128
K_GAL = 20000

TAU = 1.75
ALPHA = 0.4
S_SCALE = math.sqrt(math.log2(math.e)) / (TAU * TAU)
S_EPS = S_SCALE * 1e-12
C_THIRD = -math.log2(3.0)
T_CUT = math.log2(ALPHA) + C_THIRD

KBLK = 2048


def _main_kernel(x_ref, w1_ref, w2_ref, w3_ref, g1_ref, g2_ref, g3_ref,
                 o_ref, e1s_s, qb_s, qc_s):
    @pl.when(pl.program_id(0) == 0)
    def _embed():
        x = x_ref[...]
        e1 = jax.lax.dot_general(
            x, w1_ref[...], (((1,), (0,)), ((), ())),
            preferred_element_type=jnp.float32)
        q2 = jnp.sum(e1 * e1, axis=1, keepdims=True)
        e1s_s[...] = e1 * (-2.0 * S_SCALE)
        qb_s[...] = S_SCALE * q2 + S_EPS
        e2 = jax.lax.dot_general(
            x, w2_ref[...], (((1,), (0,)), ((), ())),
            preferred_element_type=jnp.float32)
        e3 = jax.lax.dot_general(
            x, w3_ref[...], (((1,), (0,)), ((), ())),
            preferred_element_type=jnp.float32)
        qn2 = e2 * ((1.0 / 3.0) / (jnp.sqrt(jnp.sum(e2 * e2, axis=1, keepdims=True)) + 1e-12))
        qn3 = e3 * ((1.0 / 3.0) / (jnp.sqrt(jnp.sum(e3 * e3, axis=1, keepdims=True)) + 1e-12))
        qc_s[...] = jnp.concatenate([qn2, qn3], axis=1)

    e1s = e1s_s[...]
    qb = qb_s[...]
    qc = qc_s[...]
    half = KBLK // 2
    for c in range(2):
        sl = pl.ds(c * half, half)
        g1 = g1_ref[sl, :]
        g1b = S_SCALE * jnp.sum(g1 * g1, axis=1)[None, :]
        m0 = jax.lax.dot_general(
            e1s, g1, (((1,), (1,)), ((), ())),
            preferred_element_type=jnp.float32)
        w = m0 + qb + g1b
        t = C_THIRD - w * w
        cer3 = jnp.where(t >= T_CUT, jnp.exp2(t), 0.0)

        g2 = g2_ref[sl, :]
        g3 = g3_ref[sl, :]
        r2 = 1.0 / (jnp.sqrt(jnp.sum(g2 * g2, axis=1, keepdims=True)) + 1e-12)
        r3 = 1.0 / (jnp.sqrt(jnp.sum(g3 * g3, axis=1, keepdims=True)) + 1e-12)
        gc = jnp.concatenate([g2 * r2, g3 * r3], axis=1)
        ccos = jax.lax.dot_general(
            qc, gc, (((1,), (1,)), ((), ())),
            preferred_element_type=jnp.float32)
        o_ref[:, sl] = cer3 + ccos


@functools.partial(jax.jit, static_argnames=("interpret",))
def kernel(x, W1, W2, W3, G1, G2, G3, interpret=False):
    nblk = pl.cdiv(K_GAL, KBLK)
    gal_spec = pl.BlockSpec((KBLK, D_EMB), lambda k: (k, 0))
    const2d = lambda shape: pl.BlockSpec(shape, lambda k: (0, 0))
    out = pl.pallas_call(
        _main_kernel,
        grid=(nblk,),
        in_specs=[
            const2d((Q, D_IN)),
            const2d((D_IN, D_EMB)),
            const2d((D_IN, D_EMB)),
            const2d((D_IN, D_EMB)),
            gal_spec, gal_spec, gal_spec,
        ],
        out_specs=pl.BlockSpec((Q, KBLK), lambda k: (0, k)),
        out_shape=jax.ShapeDtypeStruct((Q, K_GAL), jnp.float32),
        scratch_shapes=[
            pltpu.VMEM((Q, D_EMB), jnp.float32),
            pltpu.VMEM((Q, 1), jnp.float32),
            pltpu.VMEM((Q, 2 * D_EMB), jnp.float32),
        ],
        interpret=interpret,
    )(x, W1, W2, W3, G1, G2, G3)
    return out


# speedup vs baseline: 1.0377x; 1.0377x over previous
import functools
import math

import jax
import jax.numpy as jnp
from jax.experimental import pallas as pl
from jax.experimental.pallas import tpu as pltpu

Q = 1024
D_IN = 512
D_EMB = 128
K_GAL = 20000

TAU = 1.75
ALPHA = 0.4
S_SCALE = math.sqrt(math.log2(math.e)) / (TAU * TAU)
S_EPS = S_SCALE * 1e-12
C_THIRD = -math.log2(3.0)
T_CUT = math.log2(ALPHA) + C_THIRD

KBLK = 2048


def _main_kernel(x_ref, w1_ref, w2_ref, w3_ref, g1_ref, g2_ref, g3_ref,
                 o_ref, e1s_s, qb_s, qc_s):
    @pl.when(pl.program_id(0) == 0)
    def _embed():
        x = x_ref[...]
        e1 = jax.lax.dot_general(
            x, w1_ref[...], (((1,), (0,)), ((), ())),
            preferred_element_type=jnp.float32)
        q2 = jnp.sum(e1 * e1, axis=1, keepdims=True)
        e1s_s[...] = e1 * (-2.0 * S_SCALE)
        qb_s[...] = S_SCALE * q2 + S_EPS
        e2 = jax.lax.dot_general(
            x, w2_ref[...], (((1,), (0,)), ((), ())),
            preferred_element_type=jnp.float32)
        e3 = jax.lax.dot_general(
            x, w3_ref[...], (((1,), (0,)), ((), ())),
            preferred_element_type=jnp.float32)
        qn2 = e2 * ((1.0 / 3.0) / (jnp.sqrt(jnp.sum(e2 * e2, axis=1, keepdims=True)) + 1e-12))
        qn3 = e3 * ((1.0 / 3.0) / (jnp.sqrt(jnp.sum(e3 * e3, axis=1, keepdims=True)) + 1e-12))
        qc_s[...] = jnp.concatenate([qn2, qn3], axis=1)

    g1 = g1_ref[...]
    g1b = S_SCALE * jnp.sum(g1 * g1, axis=1)[None, :]
    m0 = jax.lax.dot_general(
        e1s_s[...], g1, (((1,), (1,)), ((), ())),
        preferred_element_type=jnp.float32)
    w = m0 + qb_s[...] + g1b
    t = C_THIRD - w * w
    cer3 = jnp.where(t >= T_CUT, jnp.exp2(t), 0.0)

    g2 = g2_ref[...]
    g3 = g3_ref[...]
    r2 = 1.0 / (jnp.sqrt(jnp.sum(g2 * g2, axis=1, keepdims=True)) + 1e-12)
    r3 = 1.0 / (jnp.sqrt(jnp.sum(g3 * g3, axis=1, keepdims=True)) + 1e-12)
    gc = jnp.concatenate([g2 * r2, g3 * r3], axis=1)
    ccos = jax.lax.dot_general(
        qc_s[...], gc, (((1,), (1,)), ((), ())),
        preferred_element_type=jnp.float32)
    o_ref[...] = cer3 + ccos


@functools.partial(jax.jit, static_argnames=("interpret",))
def kernel(x, W1, W2, W3, G1, G2, G3, interpret=False):
    nblk = pl.cdiv(K_GAL, KBLK)
    gal_spec = pl.BlockSpec((KBLK, D_EMB), lambda k: (k, 0))
    const2d = lambda shape: pl.BlockSpec(shape, lambda k: (0, 0))
    out = pl.pallas_call(
        _main_kernel,
        grid=(nblk,),
        in_specs=[
            const2d((Q, D_IN)),
            const2d((D_IN, D_EMB)),
            const2d((D_IN, D_EMB)),
            const2d((D_IN, D_EMB)),
            gal_spec, gal_spec, gal_spec,
        ],
        out_specs=pl.BlockSpec((Q, KBLK), lambda k: (0, k)),
        out_shape=jax.ShapeDtypeStruct((Q, K_GAL), jnp.float32),
        scratch_shapes=[
            pltpu.VMEM((Q, D_EMB), jnp.float32),
            pltpu.VMEM((Q, 1), jnp.float32),
            pltpu.VMEM((Q, 2 * D_EMB), jnp.float32),
        ],
        interpret=interpret,
    )(x, W1, W2, W3, G1, G2, G3)
    return out


# speedup vs baseline: 1.0406x; 1.0028x over previous
import math

import jax
import jax.numpy as jnp
from jax.experimental import pallas as pl
from jax.experimental.pallas import tpu as pltpu

Q = 1024
D_IN = 512
D_EMB = 128
K_GAL = 20000

TAU = 1.75
ALPHA = 0.4
S_SCALE = math.sqrt(math.log2(math.e)) / (TAU * TAU)
S_EPS = S_SCALE * 1e-12
C_THIRD = -math.log2(3.0)
T_CUT = math.log2(ALPHA) + C_THIRD

KBLK = 2048


def _main_kernel(x_ref, w1_ref, w2_ref, w3_ref, g1_ref, g2_ref, g3_ref,
                 o_ref, e1s_s, qb_s, qc_s):
    @pl.when(pl.program_id(0) == 0)
    def _embed():
        x = x_ref[...]
        e1 = jax.lax.dot_general(
            x, w1_ref[...], (((1,), (0,)), ((), ())),
            preferred_element_type=jnp.float32)
        q2 = jnp.sum(e1 * e1, axis=1, keepdims=True)
        e1s_s[...] = e1 * (-2.0 * S_SCALE)
        qb_s[...] = S_SCALE * q2 + S_EPS
        e2 = jax.lax.dot_general(
            x, w2_ref[...], (((1,), (0,)), ((), ())),
            preferred_element_type=jnp.float32)
        e3 = jax.lax.dot_general(
            x, w3_ref[...], (((1,), (0,)), ((), ())),
            preferred_element_type=jnp.float32)
        qn2 = e2 * ((1.0 / 3.0) / (jnp.sqrt(jnp.sum(e2 * e2, axis=1, keepdims=True)) + 1e-12))
        qn3 = e3 * ((1.0 / 3.0) / (jnp.sqrt(jnp.sum(e3 * e3, axis=1, keepdims=True)) + 1e-12))
        qc_s[...] = jnp.concatenate([qn2, qn3], axis=1)

    g1 = g1_ref[...]
    g1b = S_SCALE * jnp.sum(g1 * g1, axis=1)[None, :]
    m0 = jax.lax.dot_general(
        e1s_s[...], g1, (((1,), (1,)), ((), ())),
        preferred_element_type=jnp.float32)
    w = m0 + qb_s[...] + g1b
    t = C_THIRD - w * w
    cer3 = jnp.where(t >= T_CUT, jnp.exp2(t), 0.0)

    g2 = g2_ref[...]
    g3 = g3_ref[...]
    r2 = 1.0 / (jnp.sqrt(jnp.sum(g2 * g2, axis=1, keepdims=True)) + 1e-12)
    r3 = 1.0 / (jnp.sqrt(jnp.sum(g3 * g3, axis=1, keepdims=True)) + 1e-12)
    gc = jnp.concatenate([g2 * r2, g3 * r3], axis=1)
    ccos = jax.lax.dot_general(
        qc_s[...], gc, (((1,), (1,)), ((), ())),
        preferred_element_type=jnp.float32)
    o_ref[...] = cer3 + ccos


@jax.jit
def kernel(x, W1, W2, W3, G1, G2, G3):
    nblk = pl.cdiv(K_GAL, KBLK)
    gal_spec = pl.BlockSpec((KBLK, D_EMB), lambda k: (k, 0))
    const2d = lambda shape: pl.BlockSpec(shape, lambda k: (0, 0))
    out = pl.pallas_call(
        _main_kernel,
        grid=(nblk,),
        in_specs=[
            const2d((Q, D_IN)),
            const2d((D_IN, D_EMB)),
            const2d((D_IN, D_EMB)),
            const2d((D_IN, D_EMB)),
            gal_spec, gal_spec, gal_spec,
        ],
        out_specs=pl.BlockSpec((Q, KBLK), lambda k: (0, k)),
        out_shape=jax.ShapeDtypeStruct((Q, K_GAL), jnp.float32),
        scratch_shapes=[
            pltpu.VMEM((Q, D_EMB), jnp.float32),
            pltpu.VMEM((Q, 1), jnp.float32),
            pltpu.VMEM((Q, 2 * D_EMB), jnp.float32),
        ],
    )(x, W1, W2, W3, G1, G2, G3)
    return out
